# Initial kernel scaffold; baseline (speedup 1.0000x reference)
import jax
import jax.numpy as jnp
from jax.experimental import pallas as pl


def kernel(x, ordinal_weights):
    raise NotImplementedError("write your pallas kernel here")


# speedup vs baseline: 6.4168x; 6.4168x over previous
import functools

import jax
import jax.numpy as jnp
from jax import lax
from jax.experimental import pallas as pl
from jax.experimental.pallas import tpu as pltpu
from jax.experimental.pallas import tpu_sc as plsc

L = 16
HW = 576
NV = HW // L
N = 32
C = 768
NW = 32
CPW = C // NW

_INF = None


def _next_pow2(n):
    p = 1
    while p < n:
        p *= 2
    return p


def _ce(a, b):
    if a is _INF and b is _INF:
        return _INF, _INF
    if a is _INF:
        return b, _INF
    if b is _INF:
        return a, _INF
    return jnp.minimum(a, b), jnp.maximum(a, b)


def _bitonic_merge(S):
    n = len(S)
    if n == 1:
        return [S[0] if S[0] is _INF else jnp.sort(S[0])]
    half = n // 2
    lo, hi = [], []
    for i in range(half):
        l, h = _ce(S[i], S[i + half])
        lo.append(l)
        hi.append(h)
    return _bitonic_merge(lo) + _bitonic_merge(hi)


def _merge(A, B):
    p = _next_pow2(max(len(A), len(B)))
    Apad = A + [_INF] * (p - len(A))
    Bpad = B + [_INF] * (p - len(B))
    revB = [_INF if v is _INF else jnp.flip(v) for v in reversed(Bpad)]
    R = _bitonic_merge(Apad + revB)
    return [v for v in R if v is not _INF]


def _sort_run(vs):
    if len(vs) == 1:
        return [jnp.sort(vs[0])]
    h = len(vs) // 2
    return _merge(_sort_run(vs[:h]), _sort_run(vs[h:]))


def _lane_reduce_sum(v, lanes):
    for k in (1, 2, 4, 8):
        v = v + v.at[lanes ^ k].get(mode="promise_in_bounds")
    return v


def _sc_body(xt_hbm, w_hbm, out_hbm, xbuf, wbuf, obuf):
    wid = lax.axis_index("s") * 2 + lax.axis_index("c")
    lanes = lax.iota(jnp.int32, L)

    def chan_body(ci, _):
        c = wid * CPW + ci
        pltpu.sync_copy(w_hbm.at[c], wbuf)
        pltpu.sync_copy(xt_hbm.at[c], xbuf)

        s = jnp.zeros((L,), jnp.float32)
        for k in range(NV):
            wv = jnp.maximum(wbuf[pl.ds(k * L, L)], 0.0)
            wbuf[pl.ds(k * L, L)] = wv
            s = s + wv
        sinv = 1.0 / _lane_reduce_sum(s, lanes)

        def n_body(n, accs):
            acc0, acc1 = accs
            base = n * HW
            vs = [xbuf[pl.ds(base + k * L, L)] for k in range(NV)]
            srt = _sort_run(vs)
            dot = srt[0] * wbuf[pl.ds(0, L)]
            for k in range(1, NV):
                dot = dot + srt[k] * wbuf[pl.ds(k * L, L)]
            r = _lane_reduce_sum(dot, lanes) * sinv
            acc0 = jnp.where((n < 16) & (lanes == n), r, acc0)
            acc1 = jnp.where((n >= 16) & (lanes == n - 16), r, acc1)
            return acc0, acc1

        z = jnp.zeros((L,), jnp.float32)
        acc0, acc1 = lax.fori_loop(0, N, n_body, (z, z))
        obuf[pl.ds(0, L)] = acc0
        obuf[pl.ds(L, L)] = acc1
        pltpu.sync_copy(obuf, out_hbm.at[c])
        return ()

    lax.fori_loop(0, CPW, chan_body, ())


@jax.jit
def _run(xt, wf):
    f = pl.kernel(
        _sc_body,
        out_type=jax.ShapeDtypeStruct((C, N), jnp.float32),
        mesh=plsc.VectorSubcoreMesh(core_axis_name="c", subcore_axis_name="s"),
        compiler_params=pltpu.CompilerParams(needs_layout_passes=False),
        scratch_types=[
            pltpu.VMEM((N * HW,), jnp.float32),
            pltpu.VMEM((HW,), jnp.float32),
            pltpu.VMEM((N,), jnp.float32),
        ],
    )
    return f(xt, wf)


def kernel(x, ordinal_weights):
    xt = jnp.transpose(x.reshape(N, HW, C), (2, 0, 1)).reshape(C, N * HW)
    wf = ordinal_weights[:, ::-1]
    out = _run(xt, wf)
    return out.T.reshape(N, 1, 1, C)


# speedup vs baseline: 6.6996x; 1.0441x over previous
import functools

import jax
import jax.numpy as jnp
from jax import lax
from jax.experimental import pallas as pl
from jax.experimental.pallas import tpu as pltpu
from jax.experimental.pallas import tpu_sc as plsc

L = 16
HW = 576
NV = HW // L
N = 32
C = 768
NW = 32
CPW = C // NW

_INF = "INF"
_NINF = "NINF"


def _next_pow2(n):
    p = 1
    while p < n:
        p *= 2
    return p


def _vsort(v, desc):
    return plsc.sort_key_val(v, v, descending=desc)[0]


def _ce(a, b):
    if isinstance(a, str) or isinstance(b, str):
        if a is _INF:
            return b, a
        if b is _INF:
            return a, b
        if a is _NINF:
            return a, b
        return b, a
    return (jnp.minimum(a[0], b[0]), None), (jnp.maximum(a[0], b[0]), None)


def _bitonic_merge(S, desc):
    n = len(S)
    if n == 1:
        v = S[0]
        if isinstance(v, str):
            return [v]
        arr, sdir = v
        if sdir == desc:
            return [v]
        return [(_vsort(arr, desc), desc)]
    half = n // 2
    lo, hi = [], []
    for i in range(half):
        l, h = _ce(S[i], S[i + half])
        lo.append(l)
        hi.append(h)
    if desc:
        return _bitonic_merge(hi, desc) + _bitonic_merge(lo, desc)
    return _bitonic_merge(lo, desc) + _bitonic_merge(hi, desc)


def _merge(A, B, desc):
    p = _next_pow2(max(len(A), len(B)))
    if not desc:
        Apad = A + [_INF] * (p - len(A))
        Bpad = B + [_NINF] * (p - len(B))
    else:
        Apad = A + [_NINF] * (p - len(A))
        Bpad = B + [_INF] * (p - len(B))
    R = _bitonic_merge(Apad + Bpad, desc)
    out = [v for v in R if not isinstance(v, str)]
    assert len(out) == len(A) + len(B)
    return out


def _sort_run(vs, desc=False):
    if len(vs) == 1:
        return [(_vsort(vs[0][0], desc), desc)]
    h = len(vs) // 2
    A = _sort_run(vs[:h], desc)
    B = _sort_run(vs[h:], not desc)
    return _merge(A, B, desc)


def _lane_reduce_sum(v, lanes):
    for k in (1, 2, 4, 8):
        v = v + v.at[lanes ^ k].get(mode="promise_in_bounds")
    return v


def _sc_body(xt_hbm, w_hbm, out_hbm, xbuf, wbuf, obuf):
    wid = lax.axis_index("s") * 2 + lax.axis_index("c")
    lanes = lax.iota(jnp.int32, L)

    def chan_body(ci, _):
        c = wid * CPW + ci
        pltpu.sync_copy(w_hbm.at[c], wbuf)
        pltpu.sync_copy(xt_hbm.at[c], xbuf)

        s = jnp.zeros((L,), jnp.float32)
        for k in range(NV):
            wv = jnp.maximum(wbuf[pl.ds(k * L, L)], 0.0)
            wbuf[pl.ds(k * L, L)] = wv
            s = s + wv
        sinv = 1.0 / _lane_reduce_sum(s, lanes)

        def n_body(n, accs):
            acc0, acc1 = accs
            base = n * HW
            vs = [(xbuf[pl.ds(base + k * L, L)], None) for k in range(NV)]
            srt = _sort_run(vs)
            dot = srt[0][0] * wbuf[pl.ds(0, L)]
            for k in range(1, NV):
                dot = dot + srt[k][0] * wbuf[pl.ds(k * L, L)]
            r = _lane_reduce_sum(dot, lanes) * sinv
            acc0 = jnp.where((n < 16) & (lanes == n), r, acc0)
            acc1 = jnp.where((n >= 16) & (lanes == n - 16), r, acc1)
            return acc0, acc1

        z = jnp.zeros((L,), jnp.float32)
        acc0, acc1 = lax.fori_loop(0, N, n_body, (z, z))
        obuf[pl.ds(0, L)] = acc0
        obuf[pl.ds(L, L)] = acc1
        pltpu.sync_copy(obuf, out_hbm.at[c])
        return ()

    lax.fori_loop(0, CPW, chan_body, ())


@jax.jit
def _run(xt, wf):
    f = pl.kernel(
        _sc_body,
        out_type=jax.ShapeDtypeStruct((C, N), jnp.float32),
        mesh=plsc.VectorSubcoreMesh(core_axis_name="c", subcore_axis_name="s"),
        compiler_params=pltpu.CompilerParams(needs_layout_passes=False),
        scratch_types=[
            pltpu.VMEM((N * HW,), jnp.float32),
            pltpu.VMEM((HW,), jnp.float32),
            pltpu.VMEM((N,), jnp.float32),
        ],
    )
    return f(xt, wf)


def kernel(x, ordinal_weights):
    xt = jnp.transpose(x.reshape(N, HW, C), (2, 0, 1)).reshape(C, N * HW)
    wf = ordinal_weights[:, ::-1]
    out = _run(xt, wf)
    return out.T.reshape(N, 1, 1, C)


# speedup vs baseline: 6.9911x; 1.0435x over previous
import functools

import jax
import jax.numpy as jnp
from jax import lax
from jax.experimental import pallas as pl
from jax.experimental.pallas import tpu as pltpu
from jax.experimental.pallas import tpu_sc as plsc

L = 16
HW = 576
NV = HW // L
N = 32
C = 768
NW = 32
CPW = C // NW

_INF = "INF"
_NINF = "NINF"


def _next_pow2(n):
    p = 1
    while p < n:
        p *= 2
    return p


def _vsort(v, desc):
    return plsc.sort_key_val(v, v, descending=desc)[0]


def _ce(a, b):
    if isinstance(a, str) or isinstance(b, str):
        if a is _INF:
            return b, a
        if b is _INF:
            return a, b
        if a is _NINF:
            return a, b
        return b, a
    return (jnp.minimum(a[0], b[0]), None), (jnp.maximum(a[0], b[0]), None)


def _bitonic_merge(S, desc):
    n = len(S)
    if n == 1:
        v = S[0]
        if isinstance(v, str):
            return [v]
        arr, sdir = v
        if sdir == desc:
            return [v]
        return [(_vsort(arr, desc), desc)]
    half = n // 2
    lo, hi = [], []
    for i in range(half):
        l, h = _ce(S[i], S[i + half])
        lo.append(l)
        hi.append(h)
    if desc:
        return _bitonic_merge(hi, desc) + _bitonic_merge(lo, desc)
    return _bitonic_merge(lo, desc) + _bitonic_merge(hi, desc)


def _merge(A, B, desc):
    p = _next_pow2(max(len(A), len(B)))
    if not desc:
        Apad = A + [_INF] * (p - len(A))
        Bpad = B + [_NINF] * (p - len(B))
    else:
        Apad = A + [_NINF] * (p - len(A))
        Bpad = B + [_INF] * (p - len(B))
    R = _bitonic_merge(Apad + Bpad, desc)
    out = [v for v in R if not isinstance(v, str)]
    assert len(out) == len(A) + len(B)
    return out


def _sort_run(vs, desc=False):
    if len(vs) == 1:
        return [(_vsort(vs[0][0], desc), desc)]
    h = len(vs) // 2
    A = _sort_run(vs[:h], desc)
    B = _sort_run(vs[h:], not desc)
    return _merge(A, B, desc)


def _lane_reduce_sum(v, lanes):
    for k in (1, 2, 4, 8):
        v = v + v.at[lanes ^ k].get(mode="promise_in_bounds")
    return v


def _sc_body(xt_hbm, w_hbm, out_hbm, xbuf, wbuf, obuf):
    wid = lax.axis_index("s") * 2 + lax.axis_index("c")
    lanes = lax.iota(jnp.int32, L)

    def chan_body(ci, _):
        c = wid * CPW + ci
        pltpu.sync_copy(w_hbm.at[c], wbuf)
        pltpu.sync_copy(xt_hbm.at[c], xbuf)

        s = jnp.zeros((L,), jnp.float32)
        for k in range(NV):
            wv = jnp.maximum(wbuf[pl.ds(k * L, L)], 0.0)
            wbuf[pl.ds(k * L, L)] = wv
            s = s + wv
        sinv = 1.0 / _lane_reduce_sum(s, lanes)

        def task_dot(n):
            base = n * HW
            vs = [(xbuf[pl.ds(base + k * L, L)], None) for k in range(NV)]
            srt = _sort_run(vs)
            dot = srt[0][0] * wbuf[pl.ds(0, L)]
            for k in range(1, NV):
                dot = dot + srt[k][0] * wbuf[pl.ds(k * L, L)]
            return _lane_reduce_sum(dot, lanes) * sinv

        def n_body(n, accs):
            acc0, acc1 = accs
            ra = task_dot(n)
            rb = task_dot(n + 16)
            acc0 = jnp.where(lanes == n, ra, acc0)
            acc1 = jnp.where(lanes == n, rb, acc1)
            return acc0, acc1

        z = jnp.zeros((L,), jnp.float32)
        acc0, acc1 = lax.fori_loop(0, N // 2, n_body, (z, z))
        obuf[pl.ds(0, L)] = acc0
        obuf[pl.ds(L, L)] = acc1
        pltpu.sync_copy(obuf, out_hbm.at[c])
        return ()

    lax.fori_loop(0, CPW, chan_body, ())


@jax.jit
def _run(xt, wf):
    f = pl.kernel(
        _sc_body,
        out_type=jax.ShapeDtypeStruct((C, N), jnp.float32),
        mesh=plsc.VectorSubcoreMesh(core_axis_name="c", subcore_axis_name="s"),
        compiler_params=pltpu.CompilerParams(needs_layout_passes=False),
        scratch_types=[
            pltpu.VMEM((N * HW,), jnp.float32),
            pltpu.VMEM((HW,), jnp.float32),
            pltpu.VMEM((N,), jnp.float32),
        ],
    )
    return f(xt, wf)


def kernel(x, ordinal_weights):
    xt = jnp.transpose(x.reshape(N, HW, C), (2, 0, 1)).reshape(C, N * HW)
    wf = ordinal_weights[:, ::-1]
    out = _run(xt, wf)
    return out.T.reshape(N, 1, 1, C)
